# baseline (speedup 1.0000x reference)
import functools

import jax
import jax.numpy as jnp
from jax import lax
from jax.experimental import pallas as pl
from jax.experimental.pallas import tpu as pltpu
from jax.experimental.pallas import tpu_sc as plsc

N = 100000
D = 128
B = 1024

NC = 2
NS = 16

CHUNK = 112
RPW = 3136
NCHUNK = RPW // CHUNK
NBUF = 6
DUMMY = B
ACC_ROWS = B + 8

_f32 = jnp.float32
_i32 = jnp.int32


def _sc_body(feat_hbm, pool_hbm, seg_hbm, psums_hbm, pcnts_hbm,
             fbuf, pbuf, sbuf, ibuf, cbuf, acc, cacc, dsem, ssem):
  c = lax.axis_index("c")
  s = lax.axis_index("s")
  w = c * NS + s
  wstart = w * RPW
  wend = jnp.minimum(wstart + RPW, N)

  def in_descs(j, b):
    cs = wstart + j * CHUNK
    sj = jnp.minimum(cs, N - CHUNK)
    return (
        pltpu.make_async_copy(feat_hbm.at[pl.ds(sj, CHUNK)], fbuf.at[b],
                              dsem.at[b]),
        pltpu.make_async_copy(pool_hbm.at[pl.ds(sj, CHUNK)], pbuf.at[b],
                              dsem.at[b]),
        pltpu.make_async_copy(seg_hbm.at[pl.ds(sj, CHUNK)], sbuf.at[b],
                              dsem.at[b]),
    )

  def out_descs(b):
    return (
        pltpu.make_async_copy(fbuf.at[b], acc.at[ibuf.at[b]], ssem.at[b]),
        pltpu.make_async_copy(cbuf.at[b], cacc.at[ibuf.at[b]], ssem.at[b]),
    )

  for j in (0, 1):
    for d in in_descs(j, j % NBUF):
      d.start()

  def _zrow(r, _):
    for k in range(D // 16):
      fbuf[NBUF - 1, r, pl.ds(k * 16, 16)] = jnp.zeros((16,), _f32)
    return 0
  lax.fori_loop(0, 64, _zrow, 0)
  def _zvec(i, _):
    cbuf[NBUF - 1, pl.ds(i * 16, 16)] = jnp.zeros((16,), _f32)
    return 0
  lax.fori_loop(0, CHUNK // 16, _zvec, 0)
  pltpu.sync_copy(fbuf.at[NBUF - 1, pl.ds(0, 64)], acc.at[pl.ds(s * 64, 64)])
  pltpu.sync_copy(cbuf.at[NBUF - 1, pl.ds(0, 64)], cacc.at[pl.ds(s * 64, 64)])

  @pl.when(s == NS - 1)
  def _():
    pltpu.sync_copy(fbuf.at[NBUF - 1, pl.ds(0, 8)], acc.at[pl.ds(B, 8)])
    pltpu.sync_copy(cbuf.at[NBUF - 1, pl.ds(0, 8)], cacc.at[pl.ds(B, 8)])

  plsc.subcore_barrier()

  lane = lax.iota(_i32, 16)

  for j in range(NCHUNK):
    b = j % NBUF
    for d in in_descs(j, b):
      d.wait()

    cs = wstart + j * CHUNK
    sj = jnp.minimum(cs, N - CHUNK)

    def _grp(i, _):
      g = sj + i * 16 + lane
      m = (g >= cs) & (g < wend)
      vs = sbuf[b, pl.ds(i * 16, 16)]
      vs = jnp.minimum(jnp.maximum(vs, 0), B - 1)
      ibuf[b, pl.ds(i * 16, 16)] = jnp.where(m, vs, DUMMY)
      cbuf[b, pl.ds(i * 16, 16)] = jnp.where(m, 1.0, 0.0).astype(_f32)
      return 0
    lax.fori_loop(0, CHUNK // 16, _grp, 0)

    def _row(r2, _):
      r = r2 * 2
      ps0 = pbuf[b, pl.ds(r, 1)][0]
      ps1 = pbuf[b, pl.ds(r + 1, 1)][0]
      for k in range(D // 16):
        fbuf[b, r, pl.ds(k * 16, 16)] = fbuf[b, r, pl.ds(k * 16, 16)] * ps0
      for k in range(D // 16):
        fbuf[b, r + 1, pl.ds(k * 16, 16)] = (
            fbuf[b, r + 1, pl.ds(k * 16, 16)] * ps1)
      return 0
    lax.fori_loop(0, CHUNK // 2, _row, 0)

    for d in out_descs(b):
      d.start(add=True)

    if j + 2 < NCHUNK:
      b2 = (j + 2) % NBUF
      if j + 2 - NBUF >= 0:
        for d in out_descs(b2):
          d.wait()
      for d in in_descs(j + 2, b2):
        d.start()

  for jj in range(NCHUNK - NBUF, NCHUNK):
    for d in out_descs(jj % NBUF):
      d.wait()

  plsc.subcore_barrier()

  pltpu.sync_copy(acc.at[pl.ds(s * 64, 64)], fbuf.at[0, pl.ds(0, 64)])
  pltpu.sync_copy(fbuf.at[0, pl.ds(0, 64)], psums_hbm.at[c, pl.ds(s * 64, 64)])
  pltpu.sync_copy(cacc.at[pl.ds(s * 64, 64)], cbuf.at[0, pl.ds(0, 64)])
  pltpu.sync_copy(cbuf.at[0, pl.ds(0, 64)], pcnts_hbm.at[c, pl.ds(s * 64, 64)])


_sc_call = functools.partial(
    pl.kernel,
    out_type=(jax.ShapeDtypeStruct((NC, B, D), _f32),
              jax.ShapeDtypeStruct((NC, B), _f32)),
    mesh=plsc.VectorSubcoreMesh(core_axis_name="c", subcore_axis_name="s"),
    scratch_types=[
        pltpu.VMEM((NBUF, CHUNK, D), _f32),
        pltpu.VMEM((NBUF, CHUNK), _f32),
        pltpu.VMEM((NBUF, CHUNK), _i32),
        pltpu.VMEM((NBUF, CHUNK), _i32),
        pltpu.VMEM((NBUF, CHUNK), _f32),
        pltpu.VMEM_SHARED((ACC_ROWS, D), _f32),
        pltpu.VMEM_SHARED((ACC_ROWS,), _f32),
        pltpu.SemaphoreType.DMA((NBUF,)),
        pltpu.SemaphoreType.DMA((NBUF,)),
    ],
)(_sc_body)


def _merge_body(ps_ref, pc_ref, o_ref):
  cnt = jnp.maximum(pc_ref[0] + pc_ref[1], 1.0)
  o_ref[...] = (ps_ref[0] + ps_ref[1]) / cnt[:, None]


_merge_call = pl.pallas_call(
    _merge_body,
    out_shape=jax.ShapeDtypeStruct((B, D), _f32),
)


@jax.jit
def _run(feat, pool_ids, segment_ids):
  psums, pcnts = _sc_call(feat, pool_ids, segment_ids)
  return _merge_call(psums, pcnts)


def kernel(feat, pool_ids, segment_ids, num_segments):
  return _run(feat, pool_ids, segment_ids)


# speedup vs baseline: 1.0123x; 1.0123x over previous
import functools

import jax
import jax.numpy as jnp
from jax import lax
from jax.experimental import pallas as pl
from jax.experimental.pallas import tpu as pltpu
from jax.experimental.pallas import tpu_sc as plsc

N = 100000
D = 128
B = 1024

NC = 2
NS = 16

CHUNK = 112
RPW = 3136
NCHUNK = RPW // CHUNK
NBUF = 6
DUMMY = B
ACC_ROWS = B + 8

_f32 = jnp.float32
_i32 = jnp.int32


def _sc_body(feat_hbm, pool_hbm, seg_hbm, psums_hbm, pcnts_hbm,
             fbuf, pbuf, sbuf, ibuf, cbuf, acc, cacc, dsem, ssem):
  c = lax.axis_index("c")
  s = lax.axis_index("s")
  w = c * NS + s
  wstart = w * RPW
  wend = jnp.minimum(wstart + RPW, N)

  def in_descs(j, b):
    cs = wstart + j * CHUNK
    sj = jnp.minimum(cs, N - CHUNK)
    return (
        pltpu.make_async_copy(feat_hbm.at[pl.ds(sj, CHUNK)], fbuf.at[b],
                              dsem.at[b]),
        pltpu.make_async_copy(pool_hbm.at[pl.ds(sj, CHUNK)], pbuf.at[b],
                              dsem.at[b]),
        pltpu.make_async_copy(seg_hbm.at[pl.ds(sj, CHUNK)], sbuf.at[b],
                              dsem.at[b]),
    )

  def out_descs(b):
    return (
        pltpu.make_async_copy(fbuf.at[b], acc.at[ibuf.at[b]], ssem.at[b]),
        pltpu.make_async_copy(cbuf.at[b], cacc.at[ibuf.at[b]], ssem.at[b]),
    )

  for j in (0, 1, 2):
    for d in in_descs(j, j % NBUF):
      d.start()

  def _zrow(r, _):
    for k in range(D // 16):
      fbuf[NBUF - 1, r, pl.ds(k * 16, 16)] = jnp.zeros((16,), _f32)
    return 0
  lax.fori_loop(0, 64, _zrow, 0)
  def _zvec(i, _):
    cbuf[NBUF - 1, pl.ds(i * 16, 16)] = jnp.zeros((16,), _f32)
    return 0
  lax.fori_loop(0, CHUNK // 16, _zvec, 0)
  pltpu.sync_copy(fbuf.at[NBUF - 1, pl.ds(0, 64)], acc.at[pl.ds(s * 64, 64)])
  pltpu.sync_copy(cbuf.at[NBUF - 1, pl.ds(0, 64)], cacc.at[pl.ds(s * 64, 64)])

  @pl.when(s == NS - 1)
  def _():
    pltpu.sync_copy(fbuf.at[NBUF - 1, pl.ds(0, 8)], acc.at[pl.ds(B, 8)])
    pltpu.sync_copy(cbuf.at[NBUF - 1, pl.ds(0, 8)], cacc.at[pl.ds(B, 8)])

  plsc.subcore_barrier()

  lane = lax.iota(_i32, 16)

  for j in range(NCHUNK):
    b = j % NBUF
    for d in in_descs(j, b):
      d.wait()

    cs = wstart + j * CHUNK
    sj = jnp.minimum(cs, N - CHUNK)

    def _grp(i, _):
      g = sj + i * 16 + lane
      m = (g >= cs) & (g < wend)
      vs = sbuf[b, pl.ds(i * 16, 16)]
      vs = jnp.minimum(jnp.maximum(vs, 0), B - 1)
      ibuf[b, pl.ds(i * 16, 16)] = jnp.where(m, vs, DUMMY)
      cbuf[b, pl.ds(i * 16, 16)] = jnp.where(m, 1.0, 0.0).astype(_f32)
      return 0
    lax.fori_loop(0, CHUNK // 16, _grp, 0)

    def _row(r, _):
      ps = pbuf[b, pl.ds(r, 1)][0]
      for k in range(D // 16):
        fbuf[b, r, pl.ds(k * 16, 16)] = fbuf[b, r, pl.ds(k * 16, 16)] * ps
      return 0
    lax.fori_loop(0, CHUNK, _row, 0)

    for d in out_descs(b):
      d.start(add=True)

    if j + 3 < NCHUNK:
      b2 = (j + 3) % NBUF
      if j + 3 - NBUF >= 0:
        for d in out_descs(b2):
          d.wait()
      for d in in_descs(j + 3, b2):
        d.start()

  for jj in range(NCHUNK - NBUF, NCHUNK):
    for d in out_descs(jj % NBUF):
      d.wait()

  plsc.subcore_barrier()

  pltpu.sync_copy(acc.at[pl.ds(s * 64, 64)], fbuf.at[0, pl.ds(0, 64)])
  pltpu.sync_copy(fbuf.at[0, pl.ds(0, 64)], psums_hbm.at[c, pl.ds(s * 64, 64)])
  pltpu.sync_copy(cacc.at[pl.ds(s * 64, 64)], cbuf.at[0, pl.ds(0, 64)])
  pltpu.sync_copy(cbuf.at[0, pl.ds(0, 64)], pcnts_hbm.at[c, pl.ds(s * 64, 64)])


_sc_call = functools.partial(
    pl.kernel,
    out_type=(jax.ShapeDtypeStruct((NC, B, D), _f32),
              jax.ShapeDtypeStruct((NC, B), _f32)),
    mesh=plsc.VectorSubcoreMesh(core_axis_name="c", subcore_axis_name="s"),
    scratch_types=[
        pltpu.VMEM((NBUF, CHUNK, D), _f32),
        pltpu.VMEM((NBUF, CHUNK), _f32),
        pltpu.VMEM((NBUF, CHUNK), _i32),
        pltpu.VMEM((NBUF, CHUNK), _i32),
        pltpu.VMEM((NBUF, CHUNK), _f32),
        pltpu.VMEM_SHARED((ACC_ROWS, D), _f32),
        pltpu.VMEM_SHARED((ACC_ROWS,), _f32),
        pltpu.SemaphoreType.DMA((NBUF,)),
        pltpu.SemaphoreType.DMA((NBUF,)),
    ],
)(_sc_body)


def _merge_body(ps_ref, pc_ref, o_ref):
  cnt = jnp.maximum(pc_ref[0] + pc_ref[1], 1.0)
  o_ref[...] = (ps_ref[0] + ps_ref[1]) / cnt[:, None]


_merge_call = pl.pallas_call(
    _merge_body,
    out_shape=jax.ShapeDtypeStruct((B, D), _f32),
)


@jax.jit
def _run(feat, pool_ids, segment_ids):
  psums, pcnts = _sc_call(feat, pool_ids, segment_ids)
  return _merge_call(psums, pcnts)


def kernel(feat, pool_ids, segment_ids, num_segments):
  return _run(feat, pool_ids, segment_ids)


# speedup vs baseline: 1.0392x; 1.0266x over previous
import functools

import jax
import jax.numpy as jnp
from jax import lax
from jax.experimental import pallas as pl
from jax.experimental.pallas import tpu as pltpu
from jax.experimental.pallas import tpu_sc as plsc

N = 100000
D = 128
B = 1024

NC = 2
NS = 16

CHUNK = 112
RPW = 3136
NCHUNK = RPW // CHUNK
NBUF = 6
DUMMY = B
ACC_ROWS = B + 8

_f32 = jnp.float32
_i32 = jnp.int32


def _sc_body(feat_hbm, pool_hbm, seg_hbm, psums_hbm, pcnts_hbm,
             fbuf, pbuf, sbuf, ibuf, cbuf, acc, cacc, dsem, ssem):
  c = lax.axis_index("c")
  s = lax.axis_index("s")
  w = c * NS + s
  wstart = w * RPW
  wend = jnp.minimum(wstart + RPW, N)

  def in_descs(j, b):
    cs = wstart + j * CHUNK
    sj = jnp.minimum(cs, N - CHUNK)
    return (
        pltpu.make_async_copy(feat_hbm.at[pl.ds(sj, CHUNK)], fbuf.at[b],
                              dsem.at[b]),
        pltpu.make_async_copy(pool_hbm.at[pl.ds(sj, CHUNK)], pbuf.at[b],
                              dsem.at[b]),
        pltpu.make_async_copy(seg_hbm.at[pl.ds(sj, CHUNK)], sbuf.at[b],
                              dsem.at[b]),
    )

  def out_descs(b):
    return (
        pltpu.make_async_copy(fbuf.at[b], acc.at[ibuf.at[b]], ssem.at[b]),
        pltpu.make_async_copy(cbuf.at[b], cacc.at[ibuf.at[b]], ssem.at[b]),
    )

  for j in (0, 1):
    for d in in_descs(j, j % NBUF):
      d.start()

  def _zrow(r, _):
    for k in range(D // 16):
      fbuf[NBUF - 1, r, pl.ds(k * 16, 16)] = jnp.zeros((16,), _f32)
    return 0
  lax.fori_loop(0, 64, _zrow, 0)
  def _zvec(i, _):
    cbuf[NBUF - 1, pl.ds(i * 16, 16)] = jnp.zeros((16,), _f32)
    return 0
  lax.fori_loop(0, CHUNK // 16, _zvec, 0)
  pltpu.sync_copy(fbuf.at[NBUF - 1, pl.ds(0, 64)], acc.at[pl.ds(s * 64, 64)])
  pltpu.sync_copy(cbuf.at[NBUF - 1, pl.ds(0, 64)], cacc.at[pl.ds(s * 64, 64)])

  @pl.when(s == NS - 1)
  def _():
    pltpu.sync_copy(fbuf.at[NBUF - 1, pl.ds(0, 8)], acc.at[pl.ds(B, 8)])
    pltpu.sync_copy(cbuf.at[NBUF - 1, pl.ds(0, 8)], cacc.at[pl.ds(B, 8)])

  plsc.subcore_barrier()

  lane = lax.iota(_i32, 16)

  for j in range(NCHUNK):
    b = j % NBUF
    for d in in_descs(j, b):
      d.wait()

    cs = wstart + j * CHUNK
    sj = jnp.minimum(cs, N - CHUNK)

    def _grp(i, _):
      g = sj + i * 16 + lane
      m = (g >= cs) & (g < wend)
      vs = sbuf[b, pl.ds(i * 16, 16)]
      vs = jnp.minimum(jnp.maximum(vs, 0), B - 1)
      ibuf[b, pl.ds(i * 16, 16)] = jnp.where(m, vs, DUMMY)
      cbuf[b, pl.ds(i * 16, 16)] = jnp.where(m, 1.0, 0.0).astype(_f32)
      return 0
    lax.fori_loop(0, CHUNK // 16, _grp, 0)

    def _row(r, _):
      ps = pbuf[b, pl.ds(r, 1)][0]
      for k in range(D // 16):
        fbuf[b, r, pl.ds(k * 16, 16)] = fbuf[b, r, pl.ds(k * 16, 16)] * ps
      return 0
    lax.fori_loop(0, CHUNK, _row, 0)

    for d in out_descs(b):
      d.start(add=True)

    if j + 2 < NCHUNK:
      b2 = (j + 2) % NBUF
      if j + 2 - NBUF >= 0:
        for d in out_descs(b2):
          d.wait()
      for d in in_descs(j + 2, b2):
        d.start()

  for jj in range(NCHUNK - NBUF, NCHUNK):
    for d in out_descs(jj % NBUF):
      d.wait()

  plsc.subcore_barrier()

  pltpu.sync_copy(acc.at[pl.ds(s * 64, 64)], fbuf.at[0, pl.ds(0, 64)])
  pltpu.sync_copy(fbuf.at[0, pl.ds(0, 64)], psums_hbm.at[c, pl.ds(s * 64, 64)])
  pltpu.sync_copy(cacc.at[pl.ds(s * 64, 64)], cbuf.at[0, pl.ds(0, 64)])
  pltpu.sync_copy(cbuf.at[0, pl.ds(0, 64)], pcnts_hbm.at[c, pl.ds(s * 64, 64)])


_sc_call = functools.partial(
    pl.kernel,
    out_type=(jax.ShapeDtypeStruct((NC, B, D), _f32),
              jax.ShapeDtypeStruct((NC, B), _f32)),
    mesh=plsc.VectorSubcoreMesh(core_axis_name="c", subcore_axis_name="s"),
    scratch_types=[
        pltpu.VMEM((NBUF, CHUNK, D), _f32),
        pltpu.VMEM((NBUF, CHUNK), _f32),
        pltpu.VMEM((NBUF, CHUNK), _i32),
        pltpu.VMEM((NBUF, CHUNK), _i32),
        pltpu.VMEM((NBUF, CHUNK), _f32),
        pltpu.VMEM_SHARED((ACC_ROWS, D), _f32),
        pltpu.VMEM_SHARED((ACC_ROWS,), _f32),
        pltpu.SemaphoreType.DMA((NBUF,)),
        pltpu.SemaphoreType.DMA((NBUF,)),
    ],
)(_sc_body)


def _merge_body(ps_ref, pc_ref, o_ref):
  cnt = jnp.maximum(pc_ref[0] + pc_ref[1], 1.0)
  o_ref[...] = (ps_ref[0] + ps_ref[1]) / cnt[:, None]


_merge_call = pl.pallas_call(
    _merge_body,
    out_shape=jax.ShapeDtypeStruct((B, D), _f32),
)


@jax.jit
def _run(feat, pool_ids, segment_ids):
  psums, pcnts = _sc_call(feat, pool_ids, segment_ids)
  return _merge_call(psums, pcnts)


def kernel(feat, pool_ids, segment_ids, num_segments):
  return _run(feat, pool_ids, segment_ids)
